# baseline (speedup 1.0000x reference)
import functools

import jax
import jax.numpy as jnp
from jax import lax
from jax.experimental import pallas as pl
from jax.experimental.pallas import tpu as pltpu
from jax.experimental.pallas import tpu_sc as plsc

_S = 2048
_D = 1024
_F = 2048
_E = 8
_TM = 256
_T = _S * 2 // _TM + _E
_NPAD = _T * _TM
_NW = 32


def _router_body(w_ref, pos1_ref, pos2_ref, g1_ref, g2_ref, te_ref, act_ref):
    ww = w_ref[...]
    lane = lax.broadcasted_iota(jnp.int32, ww.shape, 1)
    m1 = jnp.max(ww, axis=-1, keepdims=True)
    a1 = jnp.argmax(ww, axis=-1)[:, None]
    masked = jnp.where(lane == a1, -jnp.inf, ww)
    m2 = jnp.max(masked, axis=-1, keepdims=True)
    a2 = jnp.argmax(masked, axis=-1)[:, None]
    denom = m1 + m2
    g1_ref[...] = m1 / denom
    g2_ref[...] = m2 / denom

    sel1 = lane == a1
    sel2 = lane == a2
    ind = jnp.where(sel1 | sel2, 1.0, 0.0).astype(jnp.bfloat16)

    row = lax.broadcasted_iota(jnp.int32, (_S, _S), 0)
    col = lax.broadcasted_iota(jnp.int32, (_S, _S), 1)
    tri = jnp.where(col < row, 1.0, 0.0).astype(jnp.bfloat16)
    cnt_before = jnp.dot(tri, ind, preferred_element_type=jnp.float32)

    counts = cnt_before[_S - 1:_S, :] + ind[_S - 1:_S, :].astype(jnp.float32)
    tiles = jnp.floor((counts + (_TM - 1.0)) * (1.0 / _TM))
    e_row = lax.broadcasted_iota(jnp.int32, (_E, _E), 0)
    e_col = lax.broadcasted_iota(jnp.int32, (_E, _E), 1)
    tri8 = jnp.where(e_row <= e_col, 1.0, 0.0)
    cum_tiles = jnp.dot(tiles, tri8, preferred_element_type=jnp.float32)
    start = (cum_tiles - tiles) * float(_TM)

    posmat = cnt_before + start
    pos1_ref[...] = jnp.sum(
        jnp.where(sel1, posmat, 0.0), axis=1, keepdims=True).astype(jnp.int32)
    pos2_ref[...] = jnp.sum(
        jnp.where(sel2, posmat, 0.0), axis=1, keepdims=True).astype(jnp.int32)

    trow = lax.broadcasted_iota(jnp.int32, (_NW, _E), 0).astype(
        jnp.float32)
    te = jnp.sum(jnp.where(trow >= cum_tiles, 1, 0), axis=1, keepdims=True)
    te_ref[...] = jnp.minimum(te, _E - 1).astype(jnp.int32)
    total = jnp.sum(tiles, axis=1, keepdims=True)
    act_ref[...] = (trow[:, :1] < total).astype(jnp.int32)


def _router(weights):
    return pl.pallas_call(
        _router_body,
        out_shape=(
            jax.ShapeDtypeStruct((_S, 1), jnp.int32),
            jax.ShapeDtypeStruct((_S, 1), jnp.int32),
            jax.ShapeDtypeStruct((_S, 1), jnp.float32),
            jax.ShapeDtypeStruct((_S, 1), jnp.float32),
            jax.ShapeDtypeStruct((_NW, 1), jnp.int32),
            jax.ShapeDtypeStruct((_NW, 1), jnp.int32),
        ),
    )(weights)


def _dispatch(x, pos_all):
    mesh = plsc.VectorSubcoreMesh(core_axis_name="c", subcore_axis_name="s")

    @functools.partial(
        pl.kernel,
        mesh=mesh,
        out_type=jax.ShapeDtypeStruct((_NPAD, _D), jnp.float32),
        scratch_types=[
            pltpu.VMEM((2, 64), jnp.int32),
            pltpu.VMEM((64, _D), jnp.float32),
        ],
    )
    def k(x_hbm, pos_hbm, xs_hbm, idx_v, rows_v):
        wid = lax.axis_index("s") * 2 + lax.axis_index("c")
        pltpu.sync_copy(pos_hbm.at[wid], idx_v)
        for h in range(2):
            tok0 = (wid % (_NW // 2)) * 128 + h * 64
            pltpu.sync_copy(x_hbm.at[pl.ds(tok0, 64)], rows_v)
            pltpu.sync_copy(rows_v, xs_hbm.at[idx_v.at[h]])

    return k(x, pos_all)


def _gmm_body(te_ref, act_ref, buf_ref, fw_ref, fi_ref, nxt_ref,
              xs_ref, w1_ref, v1_ref, w2_ref, ys_ref):
    i = pl.program_id(0)

    @pl.when(act_ref[i] == 1)
    def _():
        x = xs_ref[...]
        x1 = lax.dot_general(
            x, w1_ref[...], (((1,), (1,)), ((), ())),
            preferred_element_type=jnp.float32)
        x2 = lax.dot_general(
            x, v1_ref[...], (((1,), (1,)), ((), ())),
            preferred_element_type=jnp.float32)
        actv = x1 * lax.logistic(x1) * x2
        ys_ref[...] = jnp.dot(actv, w2_ref[...],
                              preferred_element_type=jnp.float32)


def _gmm(te_t, act_t, buf, fw, fi, nxt, xs, w1, v1, w2):
    grid_spec = pltpu.PrefetchScalarGridSpec(
        num_scalar_prefetch=6,
        grid=(_T,),
        in_specs=[
            pl.BlockSpec((_TM, _D), lambda i, *_: (i, 0)),
            pl.BlockSpec((_F, _D), lambda i, te, *_: (te[i], 0)),
            pl.BlockSpec((_F, _D), lambda i, te, *_: (te[i], 0)),
            pl.BlockSpec((_F, _D), lambda i, te, *_: (te[i], 0)),
        ],
        out_specs=pl.BlockSpec((_TM, _D), lambda i, *_: (i, 0)),
    )
    return pl.pallas_call(
        _gmm_body,
        grid_spec=grid_spec,
        compiler_params=pltpu.CompilerParams(
            dimension_semantics=("parallel",)),
        out_shape=jax.ShapeDtypeStruct((_NPAD, _D), jnp.float32),
    )(te_t, act_t, buf, fw, fi, nxt, xs, w1, v1, w2)


def _combine(ys, pos_c, g1e, g2e):
    mesh = plsc.VectorSubcoreMesh(core_axis_name="c", subcore_axis_name="s")
    tok_per_w = _S // _NW
    half = tok_per_w // 2
    _L = 16

    @functools.partial(
        pl.kernel,
        mesh=mesh,
        out_type=jax.ShapeDtypeStruct((_S, _D), jnp.float32),
        scratch_types=[
            pltpu.VMEM((half,), jnp.int32),
            pltpu.VMEM((half, _D), jnp.float32),
            pltpu.VMEM((half, _D), jnp.float32),
            pltpu.VMEM((half, _L), jnp.float32),
        ],
    )
    def k(ys_hbm, pos_hbm, g1_hbm, g2_hbm, out_hbm, idx_v, ybuf, obuf, gbuf):
        wid = lax.axis_index("s") * 2 + lax.axis_index("c")
        for h in range(2):
            base = wid * tok_per_w + h * half
            pltpu.sync_copy(pos_hbm.at[0, pl.ds(base, half)], idx_v)
            pltpu.sync_copy(ys_hbm.at[idx_v], ybuf)
            pltpu.sync_copy(g1_hbm.at[pl.ds(base, half)], gbuf)

            @pl.loop(0, half)
            def _(r):
                gv = gbuf[r, :]
                for c in range(_D // _L):
                    sl = pl.ds(c * _L, _L)
                    obuf[r, sl] = gv * ybuf[r, sl]

            pltpu.sync_copy(pos_hbm.at[1, pl.ds(base, half)], idx_v)
            pltpu.sync_copy(ys_hbm.at[idx_v], ybuf)
            pltpu.sync_copy(g2_hbm.at[pl.ds(base, half)], gbuf)

            @pl.loop(0, half)
            def _(r):
                gv = gbuf[r, :]
                for c in range(_D // _L):
                    sl = pl.ds(c * _L, _L)
                    obuf[r, sl] = obuf[r, sl] + gv * ybuf[r, sl]

            pltpu.sync_copy(obuf, out_hbm.at[pl.ds(base, half)])

    return k(ys, pos_c, g1e, g2e)


def kernel(hidden_states, router_w, w1, v1, w2):
    x = hidden_states.reshape(_S, _D)
    logits = jnp.matmul(x.astype(jnp.float32), router_w)
    weights = jax.nn.softmax(logits.astype(jnp.float32), axis=-1)

    pos1, pos2, g1, g2, te, act = _router(weights)
    pos1 = pos1[:, 0]
    pos2 = pos2[:, 0]

    pos_all = jnp.concatenate([pos1, pos2]).reshape(_NW, 2, 64)
    pos_c = jnp.stack([pos1, pos2])
    g1e = jnp.broadcast_to(g1, (_S, 16))
    g2e = jnp.broadcast_to(g2, (_S, 16))

    te_t = te[:_T, 0]
    act_t = act[:_T, 0]
    prev = jnp.concatenate([jnp.full((1,), -1, jnp.int32), te_t[:-1]])
    fw = (act_t == 1) & (te_t != prev)
    idx = jnp.arange(_T, dtype=jnp.int32)
    flag_pos = jnp.where(fw, idx, _T)
    sufmin = lax.associative_scan(jnp.minimum, flag_pos[::-1])[::-1]
    nxtpos = jnp.concatenate(
        [sufmin[1:], jnp.full((1,), _T, jnp.int32)])
    fi = fw & (nxtpos < _T)
    nxt = te_t[jnp.minimum(nxtpos, _T - 1)]
    fw = fw.astype(jnp.int32)
    fi = fi.astype(jnp.int32)
    seq = jnp.cumsum(fw) - 1
    buf = seq % 2

    xs = _dispatch(x, pos_all)
    ys = _gmm(te_t, act_t, buf, fw, fi, nxt, xs, w1, v1, w2)
    out = _combine(ys, pos_c, g1e, g2e)

    return (out.reshape(hidden_states.shape),
            weights.reshape(hidden_states.shape[0], _S, _E))


# speedup vs baseline: 1.1095x; 1.1095x over previous
import functools

import jax
import jax.numpy as jnp
from jax import lax
from jax.experimental import pallas as pl
from jax.experimental.pallas import tpu as pltpu
from jax.experimental.pallas import tpu_sc as plsc

_S = 2048
_D = 1024
_F = 2048
_E = 8
_TM = 256
_T = _S * 2 // _TM + _E
_NPAD = _T * _TM
_NW = 32


def _router_body(w_ref, pos1_ref, pos2_ref, g1_ref, g2_ref, te_ref, act_ref):
    ww = w_ref[...]
    lane = lax.broadcasted_iota(jnp.int32, ww.shape, 1)
    m1 = jnp.max(ww, axis=-1, keepdims=True)
    a1 = jnp.argmax(ww, axis=-1)[:, None]
    masked = jnp.where(lane == a1, -jnp.inf, ww)
    m2 = jnp.max(masked, axis=-1, keepdims=True)
    a2 = jnp.argmax(masked, axis=-1)[:, None]
    denom = m1 + m2
    g1_ref[...] = m1 / denom
    g2_ref[...] = m2 / denom

    sel1 = lane == a1
    sel2 = lane == a2
    ind = jnp.where(sel1 | sel2, 1.0, 0.0).astype(jnp.bfloat16)

    row = lax.broadcasted_iota(jnp.int32, (_S, _S), 0)
    col = lax.broadcasted_iota(jnp.int32, (_S, _S), 1)
    tri = jnp.where(col < row, 1.0, 0.0).astype(jnp.bfloat16)
    cnt_before = jnp.dot(tri, ind, preferred_element_type=jnp.float32)

    counts = cnt_before[_S - 1:_S, :] + ind[_S - 1:_S, :].astype(jnp.float32)
    tiles = jnp.floor((counts + (_TM - 1.0)) * (1.0 / _TM))
    e_row = lax.broadcasted_iota(jnp.int32, (_E, _E), 0)
    e_col = lax.broadcasted_iota(jnp.int32, (_E, _E), 1)
    tri8 = jnp.where(e_row <= e_col, 1.0, 0.0)
    cum_tiles = jnp.dot(tiles, tri8, preferred_element_type=jnp.float32)
    start = (cum_tiles - tiles) * float(_TM)

    posmat = cnt_before + start
    pos1_ref[...] = jnp.sum(
        jnp.where(sel1, posmat, 0.0), axis=1, keepdims=True).astype(jnp.int32)
    pos2_ref[...] = jnp.sum(
        jnp.where(sel2, posmat, 0.0), axis=1, keepdims=True).astype(jnp.int32)

    trow = lax.broadcasted_iota(jnp.int32, (_NW, _E), 0).astype(
        jnp.float32)
    te = jnp.sum(jnp.where(trow >= cum_tiles, 1, 0), axis=1, keepdims=True)
    te_ref[...] = jnp.minimum(te, _E - 1).astype(jnp.int32)
    total = jnp.sum(tiles, axis=1, keepdims=True)
    act_ref[...] = (trow[:, :1] < total).astype(jnp.int32)


def _router(weights):
    return pl.pallas_call(
        _router_body,
        out_shape=(
            jax.ShapeDtypeStruct((_S, 1), jnp.int32),
            jax.ShapeDtypeStruct((_S, 1), jnp.int32),
            jax.ShapeDtypeStruct((_S, 1), jnp.float32),
            jax.ShapeDtypeStruct((_S, 1), jnp.float32),
            jax.ShapeDtypeStruct((_NW, 1), jnp.int32),
            jax.ShapeDtypeStruct((_NW, 1), jnp.int32),
        ),
    )(weights)


def _dispatch(x, pos_all):
    mesh = plsc.VectorSubcoreMesh(core_axis_name="c", subcore_axis_name="s")

    nc = 4

    @functools.partial(
        pl.kernel,
        mesh=mesh,
        out_type=jax.ShapeDtypeStruct((_NPAD, _D), jnp.float32),
        scratch_types=[
            pltpu.VMEM((nc, 32), jnp.int32),
            pltpu.VMEM((2, 32, _D), jnp.float32),
            pltpu.SemaphoreType.DMA((2,)),
            pltpu.SemaphoreType.DMA((2,)),
        ],
    )
    def k(x_hbm, pos_hbm, xs_hbm, idx_v, rows_v, ldsem, stsem):
        wid = lax.axis_index("s") * 2 + lax.axis_index("c")
        pltpu.sync_copy(pos_hbm.at[wid], idx_v)
        tok0 = (wid % (_NW // 2)) * 128

        def ld(c):
            return pltpu.make_async_copy(
                x_hbm.at[pl.ds(tok0 + 32 * c, 32)], rows_v.at[c % 2],
                ldsem.at[c % 2])

        def st(c):
            return pltpu.make_async_copy(
                rows_v.at[c % 2], xs_hbm.at[idx_v.at[c]], stsem.at[c % 2])

        ld(0).start()
        ld(1).start()
        for c in range(nc):
            ld(c).wait()
            st(c).start()
            st(c).wait()
            if c + 2 < nc:
                ld(c + 2).start()

    return k(x, pos_all)


def _gmm_body(te_ref, act_ref, buf_ref, fw_ref, fi_ref, nxt_ref,
              xs_ref, w1_hbm, v1_hbm, w2_hbm, ys_ref,
              w1b, v1b, w2b, sems):
    i = pl.program_id(0)
    bi = buf_ref[i]

    def _descs(e, b):
        return (
            pltpu.make_async_copy(
                w1_hbm.at[pl.ds(e * _F, _F), :], w1b.at[b], sems.at[b]),
            pltpu.make_async_copy(
                v1_hbm.at[pl.ds(e * _F, _F), :], v1b.at[b], sems.at[b]),
            pltpu.make_async_copy(
                w2_hbm.at[pl.ds(e * _F, _F), :], w2b.at[b], sems.at[b]),
        )

    @pl.when(i == 0)
    def _():
        for d in _descs(te_ref[0], bi):
            d.start()

    @pl.when(fw_ref[i] == 1)
    def _():
        for d in _descs(te_ref[i], bi):
            d.wait()

    @pl.when(fi_ref[i] == 1)
    def _():
        for d in _descs(nxt_ref[i], 1 - bi):
            d.start()

    @pl.when(act_ref[i] == 1)
    def _():
        x = xs_ref[...]
        x1 = lax.dot_general(
            x, w1b[bi], (((1,), (1,)), ((), ())),
            preferred_element_type=jnp.float32)
        x2 = lax.dot_general(
            x, v1b[bi], (((1,), (1,)), ((), ())),
            preferred_element_type=jnp.float32)
        actv = x1 * lax.logistic(x1) * x2
        ys_ref[...] = jnp.dot(actv, w2b[bi],
                              preferred_element_type=jnp.float32)


def _gmm(te_t, act_t, buf, fw, fi, nxt, xs, w1, v1, w2):
    grid_spec = pltpu.PrefetchScalarGridSpec(
        num_scalar_prefetch=6,
        grid=(_T,),
        in_specs=[
            pl.BlockSpec((_TM, _D), lambda i, *_: (i, 0)),
            pl.BlockSpec(memory_space=pl.ANY),
            pl.BlockSpec(memory_space=pl.ANY),
            pl.BlockSpec(memory_space=pl.ANY),
        ],
        out_specs=pl.BlockSpec((_TM, _D), lambda i, *_: (i, 0)),
        scratch_shapes=[
            pltpu.VMEM((2, _F, _D), jnp.float32),
            pltpu.VMEM((2, _F, _D), jnp.float32),
            pltpu.VMEM((2, _F, _D), jnp.float32),
            pltpu.SemaphoreType.DMA((2,)),
        ],
    )
    return pl.pallas_call(
        _gmm_body,
        grid_spec=grid_spec,
        out_shape=jax.ShapeDtypeStruct((_NPAD, _D), jnp.float32),
    )(te_t, act_t, buf, fw, fi, nxt, xs, w1, v1, w2)


def _combine(ys, pos_c, g1e, g2e):
    mesh = plsc.VectorSubcoreMesh(core_axis_name="c", subcore_axis_name="s")
    tok_per_w = _S // _NW
    half = tok_per_w // 2
    _L = 16

    @functools.partial(
        pl.kernel,
        mesh=mesh,
        out_type=jax.ShapeDtypeStruct((_S, _D), jnp.float32),
        scratch_types=[
            pltpu.VMEM((half,), jnp.int32),
            pltpu.VMEM((half, _D), jnp.float32),
            pltpu.VMEM((half, _D), jnp.float32),
            pltpu.VMEM((half, _L), jnp.float32),
        ],
    )
    def k(ys_hbm, pos_hbm, g1_hbm, g2_hbm, out_hbm, idx_v, ybuf, obuf, gbuf):
        wid = lax.axis_index("s") * 2 + lax.axis_index("c")
        for h in range(2):
            base = wid * tok_per_w + h * half
            pltpu.sync_copy(pos_hbm.at[0, pl.ds(base, half)], idx_v)
            pltpu.sync_copy(ys_hbm.at[idx_v], ybuf)
            pltpu.sync_copy(g1_hbm.at[pl.ds(base, half)], gbuf)

            @pl.loop(0, half)
            def _(r):
                gv = gbuf[r, :]
                for c in range(_D // _L):
                    sl = pl.ds(c * _L, _L)
                    obuf[r, sl] = gv * ybuf[r, sl]

            pltpu.sync_copy(pos_hbm.at[1, pl.ds(base, half)], idx_v)
            pltpu.sync_copy(ys_hbm.at[idx_v], ybuf)
            pltpu.sync_copy(g2_hbm.at[pl.ds(base, half)], gbuf)

            @pl.loop(0, half)
            def _(r):
                gv = gbuf[r, :]
                for c in range(_D // _L):
                    sl = pl.ds(c * _L, _L)
                    obuf[r, sl] = obuf[r, sl] + gv * ybuf[r, sl]

            pltpu.sync_copy(obuf, out_hbm.at[pl.ds(base, half)])

    return k(ys, pos_c, g1e, g2e)


def kernel(hidden_states, router_w, w1, v1, w2):
    x = hidden_states.reshape(_S, _D)
    logits = jnp.matmul(x.astype(jnp.float32), router_w)
    weights = jax.nn.softmax(logits.astype(jnp.float32), axis=-1)

    pos1, pos2, g1, g2, te, act = _router(weights)
    pos1 = pos1[:, 0]
    pos2 = pos2[:, 0]

    pos_all = jnp.concatenate([pos1, pos2]).reshape(_NW, 4, 32)
    pos_c = jnp.stack([pos1, pos2])
    g1e = jnp.broadcast_to(g1, (_S, 16))
    g2e = jnp.broadcast_to(g2, (_S, 16))

    te_t = te[:_T, 0]
    act_t = act[:_T, 0]
    prev = jnp.concatenate([jnp.full((1,), -1, jnp.int32), te_t[:-1]])
    fw = (act_t == 1) & (te_t != prev)
    idx = jnp.arange(_T, dtype=jnp.int32)
    flag_pos = jnp.where(fw, idx, _T)
    sufmin = lax.associative_scan(jnp.minimum, flag_pos[::-1])[::-1]
    nxtpos = jnp.concatenate(
        [sufmin[1:], jnp.full((1,), _T, jnp.int32)])
    fi = fw & (nxtpos < _T)
    nxt = te_t[jnp.minimum(nxtpos, _T - 1)]
    fw = fw.astype(jnp.int32)
    fi = fi.astype(jnp.int32)
    seq = jnp.cumsum(fw) - 1
    buf = seq % 2

    xs = _dispatch(x, pos_all)
    ys = _gmm(te_t, act_t, buf, fw, fi, nxt, xs, w1, v1, w2)
    out = _combine(ys, pos_c, g1e, g2e)

    return (out.reshape(hidden_states.shape),
            weights.reshape(hidden_states.shape[0], _S, _E))


# speedup vs baseline: 1.1388x; 1.0264x over previous
import functools

import jax
import jax.numpy as jnp
from jax import lax
from jax.experimental import pallas as pl
from jax.experimental.pallas import tpu as pltpu
from jax.experimental.pallas import tpu_sc as plsc

_S = 2048
_D = 1024
_F = 2048
_E = 8
_TM = 256
_T = _S * 2 // _TM + _E
_NPAD = _T * _TM
_NW = 32


def _router_body(w_ref, pos1_ref, pos2_ref, g1_ref, g2_ref, te_ref, act_ref):
    ww = w_ref[...]
    lane = lax.broadcasted_iota(jnp.int32, ww.shape, 1)
    m1 = jnp.max(ww, axis=-1, keepdims=True)
    a1 = jnp.argmax(ww, axis=-1)[:, None]
    masked = jnp.where(lane == a1, -jnp.inf, ww)
    m2 = jnp.max(masked, axis=-1, keepdims=True)
    a2 = jnp.argmax(masked, axis=-1)[:, None]
    denom = m1 + m2
    g1_ref[...] = m1 / denom
    g2_ref[...] = m2 / denom

    sel1 = lane == a1
    sel2 = lane == a2
    ind = jnp.where(sel1 | sel2, 1.0, 0.0).astype(jnp.bfloat16)

    row = lax.broadcasted_iota(jnp.int32, (_S, _S), 0)
    col = lax.broadcasted_iota(jnp.int32, (_S, _S), 1)
    tri = jnp.where(col < row, 1.0, 0.0).astype(jnp.bfloat16)
    cnt_before = jnp.dot(tri, ind, preferred_element_type=jnp.float32)

    counts = cnt_before[_S - 1:_S, :] + ind[_S - 1:_S, :].astype(jnp.float32)
    tiles = jnp.floor((counts + (_TM - 1.0)) * (1.0 / _TM))
    e_row = lax.broadcasted_iota(jnp.int32, (_E, _E), 0)
    e_col = lax.broadcasted_iota(jnp.int32, (_E, _E), 1)
    tri8 = jnp.where(e_row <= e_col, 1.0, 0.0)
    cum_tiles = jnp.dot(tiles, tri8, preferred_element_type=jnp.float32)
    start = (cum_tiles - tiles) * float(_TM)

    posmat = cnt_before + start
    pos1_ref[...] = jnp.sum(
        jnp.where(sel1, posmat, 0.0), axis=1, keepdims=True).astype(jnp.int32)
    pos2_ref[...] = jnp.sum(
        jnp.where(sel2, posmat, 0.0), axis=1, keepdims=True).astype(jnp.int32)

    trow = lax.broadcasted_iota(jnp.int32, (_NW, _E), 0).astype(
        jnp.float32)
    te = jnp.sum(jnp.where(trow >= cum_tiles, 1, 0), axis=1, keepdims=True)
    te_ref[...] = jnp.minimum(te, _E - 1).astype(jnp.int32)
    total = jnp.sum(tiles, axis=1, keepdims=True)
    act_ref[...] = (trow[:, :1] < total).astype(jnp.int32)


def _router(weights):
    return pl.pallas_call(
        _router_body,
        out_shape=(
            jax.ShapeDtypeStruct((_S, 1), jnp.int32),
            jax.ShapeDtypeStruct((_S, 1), jnp.int32),
            jax.ShapeDtypeStruct((_S, 1), jnp.float32),
            jax.ShapeDtypeStruct((_S, 1), jnp.float32),
            jax.ShapeDtypeStruct((_NW, 1), jnp.int32),
            jax.ShapeDtypeStruct((_NW, 1), jnp.int32),
        ),
    )(weights)


def _dispatch(x, pos_all, gall):
    mesh = plsc.VectorSubcoreMesh(core_axis_name="c", subcore_axis_name="s")

    @functools.partial(
        pl.kernel,
        mesh=mesh,
        out_type=(
            jax.ShapeDtypeStruct((_NPAD, _D), jnp.float32),
            jax.ShapeDtypeStruct((_NPAD, 128), jnp.float32),
        ),
        scratch_types=[
            pltpu.VMEM((2, 64), jnp.int32),
            pltpu.VMEM((64, _D), jnp.float32),
            pltpu.VMEM((64, 128), jnp.float32),
        ],
    )
    def k(x_hbm, pos_hbm, g_hbm, xs_hbm, gs_hbm, idx_v, rows_v, gbuf):
        wid = lax.axis_index("s") * 2 + lax.axis_index("c")
        pltpu.sync_copy(pos_hbm.at[wid], idx_v)
        for h in range(2):
            tok0 = (wid % (_NW // 2)) * 128 + h * 64
            a0 = wid * 128 + h * 64
            pltpu.sync_copy(x_hbm.at[pl.ds(tok0, 64)], rows_v)
            pltpu.sync_copy(rows_v, xs_hbm.at[idx_v.at[h]])
            pltpu.sync_copy(g_hbm.at[pl.ds(a0, 64)], gbuf)
            pltpu.sync_copy(gbuf, gs_hbm.at[idx_v.at[h]])

    return k(x, pos_all, gall)


def _gmm_body(te_ref, act_ref, buf_ref, fw_ref, fi_ref, nxt_ref,
              xs_ref, gs_ref, w1_hbm, v1_hbm, w2_hbm, ys_ref,
              w1b, v1b, w2b, sems):
    i = pl.program_id(0)
    bi = buf_ref[i]

    def _descs(e, b):
        return (
            pltpu.make_async_copy(
                w1_hbm.at[pl.ds(e * _F, _F), :], w1b.at[b], sems.at[b]),
            pltpu.make_async_copy(
                v1_hbm.at[pl.ds(e * _F, _F), :], v1b.at[b], sems.at[b]),
            pltpu.make_async_copy(
                w2_hbm.at[pl.ds(e * _F, _F), :], w2b.at[b], sems.at[b]),
        )

    @pl.when(i == 0)
    def _():
        for d in _descs(te_ref[0], bi):
            d.start()

    @pl.when(fw_ref[i] == 1)
    def _():
        for d in _descs(te_ref[i], bi):
            d.wait()

    @pl.when(fi_ref[i] == 1)
    def _():
        for d in _descs(nxt_ref[i], 1 - bi):
            d.start()

    @pl.when(act_ref[i] == 1)
    def _():
        x = xs_ref[...]
        x1 = lax.dot_general(
            x, w1b[bi], (((1,), (1,)), ((), ())),
            preferred_element_type=jnp.float32)
        x2 = lax.dot_general(
            x, v1b[bi], (((1,), (1,)), ((), ())),
            preferred_element_type=jnp.float32)
        actv = x1 * lax.logistic(x1) * x2
        y = jnp.dot(actv, w2b[bi], preferred_element_type=jnp.float32)
        ys_ref[...] = y * gs_ref[:, :1]


def _gmm(te_t, act_t, buf, fw, fi, nxt, xs, gs, w1, v1, w2):
    grid_spec = pltpu.PrefetchScalarGridSpec(
        num_scalar_prefetch=6,
        grid=(_T,),
        in_specs=[
            pl.BlockSpec((_TM, _D), lambda i, *_: (i, 0)),
            pl.BlockSpec((_TM, 128), lambda i, *_: (i, 0)),
            pl.BlockSpec(memory_space=pl.ANY),
            pl.BlockSpec(memory_space=pl.ANY),
            pl.BlockSpec(memory_space=pl.ANY),
        ],
        out_specs=pl.BlockSpec((_TM, _D), lambda i, *_: (i, 0)),
        scratch_shapes=[
            pltpu.VMEM((2, _F, _D), jnp.float32),
            pltpu.VMEM((2, _F, _D), jnp.float32),
            pltpu.VMEM((2, _F, _D), jnp.float32),
            pltpu.SemaphoreType.DMA((2,)),
        ],
    )
    return pl.pallas_call(
        _gmm_body,
        grid_spec=grid_spec,
        out_shape=jax.ShapeDtypeStruct((_NPAD, _D), jnp.float32),
    )(te_t, act_t, buf, fw, fi, nxt, xs, gs, w1, v1, w2)


def _combine(ys, pos_c):
    mesh = plsc.VectorSubcoreMesh(core_axis_name="c", subcore_axis_name="s")
    tok_per_w = _S // _NW
    half = tok_per_w // 2
    _L = 16

    @functools.partial(
        pl.kernel,
        mesh=mesh,
        out_type=jax.ShapeDtypeStruct((_S, _D), jnp.float32),
        scratch_types=[
            pltpu.VMEM((half,), jnp.int32),
            pltpu.VMEM((half, _D), jnp.float32),
            pltpu.VMEM((half, _D), jnp.float32),
        ],
    )
    def k(ys_hbm, pos_hbm, out_hbm, idx_v, ybuf, obuf):
        wid = lax.axis_index("s") * 2 + lax.axis_index("c")
        for h in range(2):
            base = wid * tok_per_w + h * half
            pltpu.sync_copy(pos_hbm.at[0, pl.ds(base, half)], idx_v)
            pltpu.sync_copy(ys_hbm.at[idx_v], obuf)
            pltpu.sync_copy(pos_hbm.at[1, pl.ds(base, half)], idx_v)
            pltpu.sync_copy(ys_hbm.at[idx_v], ybuf)

            @pl.loop(0, half)
            def _(r):
                for c in range(_D // _L):
                    sl = pl.ds(c * _L, _L)
                    obuf[r, sl] = obuf[r, sl] + ybuf[r, sl]

            pltpu.sync_copy(obuf, out_hbm.at[pl.ds(base, half)])

    return k(ys, pos_c)


def kernel(hidden_states, router_w, w1, v1, w2):
    x = hidden_states.reshape(_S, _D)
    logits = jnp.matmul(x.astype(jnp.float32), router_w)
    weights = jax.nn.softmax(logits.astype(jnp.float32), axis=-1)

    pos1, pos2, g1, g2, te, act = _router(weights)
    pos1 = pos1[:, 0]
    pos2 = pos2[:, 0]

    pos_all = jnp.concatenate([pos1, pos2]).reshape(_NW, 2, 64)
    pos_c = jnp.stack([pos1, pos2])
    gall = jnp.broadcast_to(
        jnp.concatenate([g1, g2]), (2 * _S, 128))

    te_t = te[:_T, 0]
    act_t = act[:_T, 0]
    prev = jnp.concatenate([jnp.full((1,), -1, jnp.int32), te_t[:-1]])
    fw = (act_t == 1) & (te_t != prev)
    idx = jnp.arange(_T, dtype=jnp.int32)
    flag_pos = jnp.where(fw, idx, _T)
    sufmin = lax.associative_scan(jnp.minimum, flag_pos[::-1])[::-1]
    nxtpos = jnp.concatenate(
        [sufmin[1:], jnp.full((1,), _T, jnp.int32)])
    fi = fw & (nxtpos < _T)
    nxt = te_t[jnp.minimum(nxtpos, _T - 1)]
    fw = fw.astype(jnp.int32)
    fi = fi.astype(jnp.int32)
    seq = jnp.cumsum(fw) - 1
    buf = seq % 2

    xs, gs = _dispatch(x, pos_all, gall)
    ys = _gmm(te_t, act_t, buf, fw, fi, nxt, xs, gs, w1, v1, w2)
    out = _combine(ys, pos_c)

    return (out.reshape(hidden_states.shape),
            weights.reshape(hidden_states.shape[0], _S, _E))
